# baseline (speedup 1.0000x reference)
import functools

import jax
import jax.numpy as jnp
from jax import lax
from jax.experimental import pallas as pl
from jax.experimental.pallas import tpu as pltpu
from jax.experimental.pallas import tpu_sc as plsc

_B = 4096
_V = 100001
_EMB = 64
_N_GROUP = 19
_N_GRAPH = 30
_N_COLOUR = 20
_D_OUT = _EMB + _N_GROUP + _N_GRAPH + _N_COLOUR
_D_OH = _D_OUT - _EMB
_BN_EPS = 1e-3

_NC = 2
_NS = 16
_L = 16
_NW = _NC * _NS
_DPW = _EMB // _NW
_APW = _B // _NW
_UNROLL = 8

_H = 50048
_H2 = 49920
_VT = _H + _H2
_NT = _V - _VT

_OFF_GROUP = _EMB
_OFF_GRAPH = _EMB + _N_GROUP
_OFF_COLOUR = _OFF_GRAPH + _N_GRAPH


@functools.partial(
    pl.kernel,
    mesh=plsc.VectorSubcoreMesh(core_axis_name="c", subcore_axis_name="s"),
    compiler_params=pltpu.CompilerParams(
        needs_layout_passes=False, use_tc_tiling_on_sc=True),
    out_type=jax.ShapeDtypeStruct((_D_OUT, _B), jnp.float32),
    scratch_types=[
        pltpu.VMEM((_B,), jnp.int32),
        pltpu.VMEM((_APW,), jnp.int32),
        pltpu.VMEM((_APW,), jnp.int32),
        pltpu.VMEM((_APW,), jnp.int32),
        pltpu.VMEM((144,), jnp.float32),
        pltpu.VMEM((144,), jnp.float32),
        pltpu.VMEM((_H,), jnp.float32),
        pltpu.VMEM((_H2,), jnp.float32),
        pltpu.VMEM((_NT, _EMB), jnp.float32),
        pltpu.VMEM((_B,), jnp.float32),
        pltpu.VMEM((_D_OH, _APW), jnp.float32),
        pltpu.SemaphoreType.DMA,
        pltpu.SemaphoreType.DMA,
        pltpu.SemaphoreType.DMA,
    ],
)
def _article_sc(table_hbm, aid_hbm, grp_hbm, gph_hbm, col_hbm, scale_hbm,
                shift_hbm, tail_hbm, out_hbm, aid_v, grp_v, gph_v, colr_v,
                scale_v, shift_v, rowa_v, rowb_v, tail_v, col_v, oh_v,
                sema, semb, osem):
    wid = lax.axis_index("s") * _NC + lax.axis_index("c")
    abase = wid * _APW
    d0 = wid * _DPW

    cpa = pltpu.async_copy(table_hbm.at[d0, pl.ds(0, _H)], rowa_v, sema)
    cpb = pltpu.async_copy(table_hbm.at[d0, pl.ds(_H, _H2)], rowb_v, semb)

    pltpu.sync_copy(tail_hbm, tail_v)
    pltpu.sync_copy(aid_hbm, aid_v)
    pltpu.sync_copy(grp_hbm.at[pl.ds(abase, _APW)], grp_v)
    pltpu.sync_copy(gph_hbm.at[pl.ds(abase, _APW)], gph_v)
    pltpu.sync_copy(col_hbm.at[pl.ds(abase, _APW)], colr_v)
    pltpu.sync_copy(scale_hbm, scale_v.at[pl.ds(0, _D_OUT)])
    pltpu.sync_copy(shift_hbm, shift_v.at[pl.ds(0, _D_OUT)])

    for r in range(_D_OH):
        sh_r = plsc.load_gather(shift_v, [jnp.full((_L,), _EMB + r,
                                                   jnp.int32)])
        for c in range(_APW // _L):
            oh_v[r, pl.ds(c * _L, _L)] = sh_r
    lane = lax.iota(jnp.int32, _L)
    for blk in range(_APW // _L):
        cols = lane + blk * _L
        for idx_ref, off in ((grp_v, _OFF_GROUP), (gph_v, _OFF_GRAPH),
                             (colr_v, _OFF_COLOUR)):
            ids = idx_ref[pl.ds(blk * _L, _L)] + (off - _EMB)
            vals = plsc.load_gather(scale_v, [ids + _EMB])
            plsc.addupdate_scatter(oh_v, [ids, cols], vals)
    ohcp = pltpu.async_copy(
        oh_v, out_hbm.at[pl.ds(_EMB, _D_OH), pl.ds(abase, _APW)], osem)

    for k in range(_DPW):
        d = d0 + k
        dsplat = jnp.full((_L,), d, jnp.int32)
        sc_d = plsc.load_gather(scale_v, [dsplat])
        sh_d = plsc.load_gather(shift_v, [dsplat])

        cpa.wait()

        def pass_a(j, carry):
            for u in range(_UNROLL):
                o = (j * _UNROLL + u) * _L
                ids = aid_v[pl.ds(o, _L)]
                ga = plsc.load_gather(rowa_v, [jnp.minimum(ids, _H - 1)])
                col_v[pl.ds(o, _L)] = jnp.where(ids < _H, ga, 0.0)
            return carry

        lax.fori_loop(0, _B // _L // _UNROLL, pass_a, 0)
        if k + 1 < _DPW:
            cpa = pltpu.async_copy(
                table_hbm.at[d + 1, pl.ds(0, _H)], rowa_v, sema)

        cpb.wait()

        def pass_b(j, carry):
            for u in range(_UNROLL):
                o = (j * _UNROLL + u) * _L
                ids = aid_v[pl.ds(o, _L)]
                gb = plsc.load_gather(
                    rowb_v,
                    [jnp.clip(ids - _H, 0, _H2 - 1)])
                gt = plsc.load_gather(
                    tail_v,
                    [jnp.maximum(ids - _VT, 0), dsplat])
                hi = jnp.where(ids < _VT, gb, gt)
                merged = jnp.where(ids < _H, col_v[pl.ds(o, _L)], hi)
                col_v[pl.ds(o, _L)] = merged * sc_d + sh_d
            return carry

        lax.fori_loop(0, _B // _L // _UNROLL, pass_b, 0)
        if k + 1 < _DPW:
            cpb = pltpu.async_copy(
                table_hbm.at[d + 1, pl.ds(_H, _H2)], rowb_v, semb)

        pltpu.sync_copy(col_v, out_hbm.at[d])

    ohcp.wait()


def kernel(article_id, product_group_name, graphical_appearance_name,
           perceived_colour_master_name, emb_table, gamma, beta,
           moving_mean, moving_var):
    scale = gamma * lax.rsqrt(moving_var + _BN_EPS)
    shift = beta - moving_mean * scale
    table_t = emb_table.T
    out_t = _article_sc(
        table_t,
        article_id.astype(jnp.int32),
        product_group_name.astype(jnp.int32),
        graphical_appearance_name.astype(jnp.int32),
        perceived_colour_master_name.astype(jnp.int32),
        scale,
        shift,
        emb_table[_VT:, :],
    )
    return out_t.T


# speedup vs baseline: 1.1242x; 1.1242x over previous
import functools

import jax
import jax.numpy as jnp
from jax import lax
from jax.experimental import pallas as pl
from jax.experimental.pallas import tpu as pltpu
from jax.experimental.pallas import tpu_sc as plsc

_B = 4096
_V = 100001
_EMB = 64
_N_GROUP = 19
_N_GRAPH = 30
_N_COLOUR = 20
_D_OUT = _EMB + _N_GROUP + _N_GRAPH + _N_COLOUR
_D_OH = _D_OUT - _EMB
_BN_EPS = 1e-3

_NC = 2
_NS = 16
_L = 16
_NW = _NC * _NS
_DPW = _EMB // _NW
_APW = _B // _NW
_UNROLL = 8

_H = 50048
_H2 = 49920
_VT = _H + _H2
_NT = _V - _VT

_OFF_GROUP = _EMB
_OFF_GRAPH = _EMB + _N_GROUP
_OFF_COLOUR = _OFF_GRAPH + _N_GRAPH


@functools.partial(
    pl.kernel,
    mesh=plsc.VectorSubcoreMesh(core_axis_name="c", subcore_axis_name="s"),
    compiler_params=pltpu.CompilerParams(
        needs_layout_passes=False, use_tc_tiling_on_sc=True),
    out_type=jax.ShapeDtypeStruct((_D_OUT, _B), jnp.float32),
    scratch_types=[
        pltpu.VMEM((_B,), jnp.int32),
        pltpu.VMEM((_APW,), jnp.int32),
        pltpu.VMEM((_APW,), jnp.int32),
        pltpu.VMEM((_APW,), jnp.int32),
        pltpu.VMEM((144,), jnp.float32),
        pltpu.VMEM((144,), jnp.float32),
        pltpu.VMEM((_V,), jnp.float32),
        pltpu.VMEM((_B,), jnp.float32),
        pltpu.VMEM((_D_OH, _APW), jnp.float32),
        pltpu.SemaphoreType.DMA,
        pltpu.SemaphoreType.DMA,
    ],
)
def _article_sc(table_hbm, aid_hbm, grp_hbm, gph_hbm, col_hbm, scale_hbm,
                shift_hbm, out_hbm, aid_v, grp_v, gph_v, colr_v,
                scale_v, shift_v, row_v, col_v, oh_v, sema, osem):
    wid = lax.axis_index("s") * _NC + lax.axis_index("c")
    abase = wid * _APW
    d0 = wid * _DPW

    rowcp = pltpu.async_copy(table_hbm.at[d0], row_v, sema)

    pltpu.sync_copy(aid_hbm, aid_v)
    pltpu.sync_copy(grp_hbm.at[pl.ds(abase, _APW)], grp_v)
    pltpu.sync_copy(gph_hbm.at[pl.ds(abase, _APW)], gph_v)
    pltpu.sync_copy(col_hbm.at[pl.ds(abase, _APW)], colr_v)
    pltpu.sync_copy(scale_hbm, scale_v.at[pl.ds(0, _D_OUT)])
    pltpu.sync_copy(shift_hbm, shift_v.at[pl.ds(0, _D_OUT)])

    for r in range(_D_OH):
        sh_r = plsc.load_gather(shift_v, [jnp.full((_L,), _EMB + r,
                                                   jnp.int32)])
        for c in range(_APW // _L):
            oh_v[r, pl.ds(c * _L, _L)] = sh_r
    lane = lax.iota(jnp.int32, _L)
    for blk in range(_APW // _L):
        cols = lane + blk * _L
        for idx_ref, off in ((grp_v, _OFF_GROUP), (gph_v, _OFF_GRAPH),
                             (colr_v, _OFF_COLOUR)):
            ids = idx_ref[pl.ds(blk * _L, _L)] + (off - _EMB)
            vals = plsc.load_gather(scale_v, [ids + _EMB])
            plsc.addupdate_scatter(oh_v, [ids, cols], vals)
    ohcp = pltpu.async_copy(
        oh_v, out_hbm.at[pl.ds(_EMB, _D_OH), pl.ds(abase, _APW)], osem)

    for k in range(_DPW):
        d = d0 + k
        dsplat = jnp.full((_L,), d, jnp.int32)
        sc_d = plsc.load_gather(scale_v, [dsplat])
        sh_d = plsc.load_gather(shift_v, [dsplat])

        rowcp.wait()

        def gath(j, carry):
            for u in range(_UNROLL):
                o = (j * _UNROLL + u) * _L
                ids = aid_v[pl.ds(o, _L)]
                col_v[pl.ds(o, _L)] = (
                    plsc.load_gather(row_v, [ids]) * sc_d + sh_d)
            return carry

        lax.fori_loop(0, _B // _L // _UNROLL, gath, 0)
        if k + 1 < _DPW:
            rowcp = pltpu.async_copy(table_hbm.at[d + 1], row_v, sema)

        pltpu.sync_copy(col_v, out_hbm.at[d])

    ohcp.wait()


def kernel(article_id, product_group_name, graphical_appearance_name,
           perceived_colour_master_name, emb_table, gamma, beta,
           moving_mean, moving_var):
    scale = gamma * lax.rsqrt(moving_var + _BN_EPS)
    shift = beta - moving_mean * scale
    table_t = emb_table.T
    out_t = _article_sc(
        table_t,
        article_id.astype(jnp.int32),
        product_group_name.astype(jnp.int32),
        graphical_appearance_name.astype(jnp.int32),
        perceived_colour_master_name.astype(jnp.int32),
        scale,
        shift,
    )
    return out_t.T
